# Initial kernel scaffold; baseline (speedup 1.0000x reference)
import jax
import jax.numpy as jnp
from jax.experimental import pallas as pl


def kernel(z, edge_index, batch, z_table, W0, b0, W1, b1, W2, b2, lin1_W, lin1_b, lin2_W, lin2_b):
    raise NotImplementedError("write your pallas kernel here")


# speedup vs baseline: 11.9127x; 11.9127x over previous
import jax
import jax.numpy as jnp
from jax import lax
from jax.experimental import pallas as pl
from jax.experimental.pallas import tpu as pltpu
from jax.experimental.pallas import tpu_sc as plsc

N = 10000
NP = 10240
E = 320000
H = 128
G = 64
NC = 2
NS = 16
CH = 128

EPC = E // NC
NCHUNK = EPC // CH
STRIPE = NP // NS
HW = 16
NZCH = NP // CH


def _sc_prep_body(z_hbm, dst_hbm, ztab_hbm, zeros_hbm, ones_hbm,
                  x_hbm, degp_hbm,
                  z_v, dst_v, ones_v, rows_v, hist_sh, sem):
    cid = lax.axis_index("c")
    sid = lax.axis_index("s")
    r0 = sid * STRIPE

    pltpu.sync_copy(zeros_hbm.at[pl.ds(r0, STRIPE)], hist_sh.at[pl.ds(r0, STRIPE)])
    pltpu.sync_copy(ones_hbm, ones_v)
    plsc.subcore_barrier()

    rem = NCHUNK % NS
    nj = (NCHUNK // NS) + jnp.where(sid < rem, 1, 0)

    def hist_body(j, carry):
        c = sid + NS * j
        base = cid * EPC + c * CH
        pltpu.sync_copy(dst_hbm.at[pl.ds(base, CH)], dst_v)
        pltpu.sync_copy(ones_v, hist_sh.at[dst_v], add=True)
        return carry

    lax.fori_loop(0, nj, hist_body, 0)

    @pl.when(cid == 0)
    def _():
        def gath_body(j, carry):
            base = (sid + NS * j) * CH
            pltpu.sync_copy(z_hbm.at[pl.ds(base, CH)], z_v)
            pltpu.async_copy(ztab_hbm.at[z_v], rows_v, sem).wait()
            pltpu.sync_copy(rows_v, x_hbm.at[pl.ds(base, CH)])
            return carry

        lax.fori_loop(0, NZCH // NS, gath_body, 0)

    plsc.subcore_barrier()
    pltpu.sync_copy(hist_sh.at[pl.ds(r0, STRIPE)],
                    degp_hbm.at[cid, pl.ds(r0, STRIPE)])


_sc_prep = pl.kernel(
    _sc_prep_body,
    out_type=(jax.ShapeDtypeStruct((NP, H), jnp.float32),
              jax.ShapeDtypeStruct((NC, NP, H), jnp.float32)),
    mesh=plsc.VectorSubcoreMesh(core_axis_name="c", subcore_axis_name="s",
                                num_cores=NC, num_subcores=NS),
    scratch_types=(
        pltpu.VMEM((CH,), jnp.int32),
        pltpu.VMEM((CH,), jnp.int32),
        pltpu.VMEM((CH, H), jnp.float32),
        pltpu.VMEM((CH, H), jnp.float32),
        pltpu.VMEM_SHARED((NP, H), jnp.float32),
        pltpu.SemaphoreType.DMA,
    ),
)


def _sc_agg_body(yp_hbm, src_hbm, dst_hbm, zeros_hbm,
                 accp_hbm,
                 src_v, dst_v, rows_v, acc_sh, sem):
    cid = lax.axis_index("c")
    sid = lax.axis_index("s")
    r0 = sid * STRIPE

    pltpu.sync_copy(zeros_hbm.at[pl.ds(r0, STRIPE)], acc_sh.at[pl.ds(r0, STRIPE)])
    plsc.subcore_barrier()

    rem = NCHUNK % NS
    nj = (NCHUNK // NS) + jnp.where(sid < rem, 1, 0)

    def body(j, carry):
        c = sid + NS * j
        base = cid * EPC + c * CH
        pltpu.sync_copy(src_hbm.at[pl.ds(base, CH)], src_v)
        pltpu.sync_copy(dst_hbm.at[pl.ds(base, CH)], dst_v)
        pltpu.async_copy(yp_hbm.at[src_v], rows_v, sem).wait()
        pltpu.sync_copy(rows_v, acc_sh.at[dst_v], add=True)
        return carry

    lax.fori_loop(0, nj, body, 0)

    plsc.subcore_barrier()
    pltpu.sync_copy(acc_sh.at[pl.ds(r0, STRIPE)],
                    accp_hbm.at[cid, pl.ds(r0, STRIPE)])


_sc_agg = pl.kernel(
    _sc_agg_body,
    out_type=jax.ShapeDtypeStruct((NC, NP, H), jnp.float32),
    mesh=plsc.VectorSubcoreMesh(core_axis_name="c", subcore_axis_name="s",
                                num_cores=NC, num_subcores=NS),
    scratch_types=(
        pltpu.VMEM((CH,), jnp.int32),
        pltpu.VMEM((CH,), jnp.int32),
        pltpu.VMEM((CH, H), jnp.float32),
        pltpu.VMEM_SHARED((NP, H), jnp.float32),
        pltpu.SemaphoreType.DMA,
    ),
)


BR = 1024


def _tc_first_body(x_ref, degp_ref, w_ref, yp_ref, dinv_ref):
    dp = degp_ref[...]
    deg = 1.0 + dp[0, :, 0:1] + dp[1, :, 0:1]
    dinv = lax.rsqrt(deg)
    y = jnp.dot(x_ref[...], w_ref[...], preferred_element_type=jnp.float32)
    yp_ref[...] = y * dinv
    dinv_ref[...] = dinv


_tc_first = pl.pallas_call(
    _tc_first_body,
    grid=(NP // BR,),
    in_specs=[
        pl.BlockSpec((BR, H), lambda i: (i, 0)),
        pl.BlockSpec((NC, BR, H), lambda i: (0, i, 0)),
        pl.BlockSpec((H, H), lambda i: (0, 0)),
    ],
    out_specs=[
        pl.BlockSpec((BR, H), lambda i: (i, 0)),
        pl.BlockSpec((BR, 1), lambda i: (i, 0)),
    ],
    out_shape=[jax.ShapeDtypeStruct((NP, H), jnp.float32),
               jax.ShapeDtypeStruct((NP, 1), jnp.float32)],
)


def _tc_mid_body(accp_ref, yp_ref, dinv_ref, b_ref, w_ref, out_ref):
    a = accp_ref[...]
    dinv = dinv_ref[...]
    x = (a[0] + a[1] + yp_ref[...]) * dinv + b_ref[...]
    x = jnp.maximum(x, 0.0)
    out_ref[...] = jnp.dot(x, w_ref[...],
                           preferred_element_type=jnp.float32) * dinv


_tc_mid = pl.pallas_call(
    _tc_mid_body,
    grid=(NP // BR,),
    in_specs=[
        pl.BlockSpec((NC, BR, H), lambda i: (0, i, 0)),
        pl.BlockSpec((BR, H), lambda i: (i, 0)),
        pl.BlockSpec((BR, 1), lambda i: (i, 0)),
        pl.BlockSpec((1, H), lambda i: (0, 0)),
        pl.BlockSpec((H, H), lambda i: (0, 0)),
    ],
    out_specs=pl.BlockSpec((BR, H), lambda i: (i, 0)),
    out_shape=jax.ShapeDtypeStruct((NP, H), jnp.float32),
)


def _tc_pool_body(accp_ref, yp_ref, dinv_ref, b2_ref, bat_ref,
                  l1w_ref, l1b_ref, l2w_ref, l2b_ref, out_ref):
    a = accp_ref[...]
    x3 = (a[0] + a[1] + yp_ref[...]) * dinv_ref[...] + b2_ref[...]

    bat = bat_ref[...]
    gids = lax.broadcasted_iota(jnp.int32, (G, 1), 0)
    m = (bat < gids).astype(jnp.float32)
    center = jnp.sum(m, axis=1, keepdims=True).astype(jnp.int32)
    cs = jnp.minimum(center, N - 1)
    cd = jnp.minimum(center + 1, N - 1)
    cols = lax.broadcasted_iota(jnp.int32, (1, NP), 1)
    ohs = (cols == cs).astype(jnp.float32)
    ohd = (cols == cd).astype(jnp.float32)
    xs = jnp.dot(ohs, x3, preferred_element_type=jnp.float32)
    xd = jnp.dot(ohd, x3, preferred_element_type=jnp.float32)
    h = xs * xd
    h = jnp.maximum(
        jnp.dot(h, l1w_ref[...], preferred_element_type=jnp.float32)
        + l1b_ref[...], 0.0)
    out_ref[...] = (jnp.dot(h, l2w_ref[...], preferred_element_type=jnp.float32)
                    + l2b_ref[...])


_tc_pool = pl.pallas_call(
    _tc_pool_body,
    out_shape=jax.ShapeDtypeStruct((G, 1), jnp.float32),
)


def kernel(z, edge_index, batch, z_table, W0, b0, W1, b1, W2, b2,
           lin1_W, lin1_b, lin2_W, lin2_b):
    z_pad = jnp.concatenate(
        [z.astype(jnp.int32), jnp.zeros((NP - N,), jnp.int32)])
    src = edge_index[0].astype(jnp.int32)
    dst = edge_index[1].astype(jnp.int32)
    bat_row = jnp.concatenate(
        [batch.astype(jnp.int32), jnp.full((NP - N,), G + 1, jnp.int32)]
    ).reshape(1, NP)

    zeros_nh = jnp.zeros((NP, H), jnp.float32)
    ones_ch = jnp.ones((CH, H), jnp.float32)

    x, degp = _sc_prep(z_pad, dst, z_table, zeros_nh, ones_ch)
    y0p, dinv = _tc_first(x, degp, W0)
    acc0 = _sc_agg(y0p, src, dst, zeros_nh)
    y1p = _tc_mid(acc0, y0p, dinv, b0.reshape(1, H), W1)
    acc1 = _sc_agg(y1p, src, dst, zeros_nh)
    y2p = _tc_mid(acc1, y1p, dinv, b1.reshape(1, H), W2)
    acc2 = _sc_agg(y2p, src, dst, zeros_nh)
    out = _tc_pool(acc2, y2p, dinv, b2.reshape(1, H), bat_row,
                   lin1_W, lin1_b.reshape(1, H), lin2_W,
                   lin2_b.reshape(1, 1))
    return out
